# Initial kernel scaffold; baseline (speedup 1.0000x reference)
import jax
import jax.numpy as jnp
from jax.experimental import pallas as pl


def kernel(tokens, table):
    raise NotImplementedError("write your pallas kernel here")


# speedup vs baseline: 2.9600x; 2.9600x over previous
import math

import jax
import jax.numpy as jnp
from jax import lax
from jax.experimental import pallas as pl
from jax.experimental.pallas import tpu as pltpu
from jax.experimental.pallas import tpu_sc as plsc

VOCAB_ = 100000
EMB_ = 128
N_TOKENS = 4096 * 50
NW = 32
PER_W = N_TOKENS // NW
CHUNK = 128
NSTEPS = PER_W // CHUNK
NBUF = 5
NGRP = NSTEPS // NBUF
SCALE = math.sqrt(EMB_)


def _body(tokens_hbm, table_hbm, out_hbm, idx_v, bufs, gsems, osems):
    nc = 2
    wid = lax.axis_index("s") * nc + lax.axis_index("c")
    obase = wid * PER_W

    pltpu.sync_copy(tokens_hbm.at[wid], idx_v)

    def gather(step, b):
        return pltpu.make_async_copy(
            table_hbm.at[idx_v.at[step]], bufs[b], gsems[b])

    def writeback(step, b):
        return pltpu.make_async_copy(
            bufs[b], out_hbm.at[pl.ds(obase + step * CHUNK, CHUNK)], osems[b])

    for b in range(NBUF):
        gather(b, b).start()

    def scale_row(r, buf):
        for j in range(EMB_ // 16):
            sl = pl.ds(j * 16, 16)
            buf[r, sl] = buf[r, sl] * SCALE
        return r + 1

    def grp_body(grp, carry):
        for b in range(NBUF):
            step = grp * NBUF + b
            gather(step, b).wait()
            lax.fori_loop(0, CHUNK, lambda r, _: scale_row(r, bufs[b]), 0)
            writeback(step, b).start()

            @pl.when(grp + 1 < NGRP)
            def _():
                writeback(step, b).wait()
                gather(step + NBUF, b).start()
        return carry

    lax.fori_loop(0, NGRP, grp_body, 0)

    for b in range(NBUF):
        writeback((NGRP - 1) * NBUF + b, b).wait()


def kernel(tokens, table):
    toks = tokens.reshape(NW, NSTEPS, CHUNK).astype(jnp.int32)
    mesh = plsc.VectorSubcoreMesh(core_axis_name="c", subcore_axis_name="s")
    out = pl.kernel(
        _body,
        out_type=jax.ShapeDtypeStruct((N_TOKENS, EMB_), jnp.float32),
        mesh=mesh,
        scratch_types=[
            pltpu.VMEM((NSTEPS, CHUNK), jnp.int32),
            [pltpu.VMEM((CHUNK, EMB_), jnp.float32) for _ in range(NBUF)],
            [pltpu.SemaphoreType.DMA for _ in range(NBUF)],
            [pltpu.SemaphoreType.DMA for _ in range(NBUF)],
        ],
    )(toks, table)
    return out.reshape(tokens.shape[0], tokens.shape[1], EMB_)


# speedup vs baseline: 2.9647x; 1.0016x over previous
import math

import jax
import jax.numpy as jnp
from jax import lax
from jax.experimental import pallas as pl
from jax.experimental.pallas import tpu as pltpu
from jax.experimental.pallas import tpu_sc as plsc

VOCAB_ = 100000
EMB_ = 128
N_TOKENS = 4096 * 50
NW = 32
PER_W = N_TOKENS // NW
CHUNK = 128
NSTEPS = PER_W // CHUNK
NBUF = 5
NGRP = NSTEPS // NBUF
REFILL_D = 2
SCALE = math.sqrt(EMB_)


def _body(tokens_hbm, table_hbm, out_hbm, idx_v, bufs, gsems, osems):
    nc = 2
    wid = lax.axis_index("s") * nc + lax.axis_index("c")
    obase = wid * PER_W

    pltpu.sync_copy(tokens_hbm.at[wid], idx_v)

    def gather(step, b):
        return pltpu.make_async_copy(
            table_hbm.at[idx_v.at[step]], bufs[b], gsems[b])

    def writeback(step, b):
        return pltpu.make_async_copy(
            bufs[b], out_hbm.at[pl.ds(obase + step * CHUNK, CHUNK)], osems[b])

    for b in range(NBUF):
        gather(b, b).start()

    def scale_rows(r, buf):
        for rr in range(2):
            for j in range(EMB_ // 16):
                sl = pl.ds(j * 16, 16)
                buf[2 * r + rr, sl] = buf[2 * r + rr, sl] * SCALE
        return r

    def grp_body(grp, carry):
        for b in range(NBUF):
            step = grp * NBUF + b
            rb = (b - REFILL_D) % NBUF
            rstep = step - REFILL_D

            @pl.when((rstep >= 0) & (rstep + NBUF < NSTEPS))
            def _():
                writeback(rstep, rb).wait()
                gather(rstep + NBUF, rb).start()

            gather(step, b).wait()
            lax.fori_loop(0, CHUNK // 2, lambda r, _: scale_rows(r, bufs[b]), 0)
            writeback(step, b).start()
        return carry

    lax.fori_loop(0, NGRP, grp_body, 0)

    for b in range(NBUF):
        writeback((NGRP - 1) * NBUF + b, b).wait()


def kernel(tokens, table):
    toks = tokens.reshape(NW, NSTEPS, CHUNK).astype(jnp.int32)
    mesh = plsc.VectorSubcoreMesh(core_axis_name="c", subcore_axis_name="s")
    out = pl.kernel(
        _body,
        out_type=jax.ShapeDtypeStruct((N_TOKENS, EMB_), jnp.float32),
        mesh=mesh,
        scratch_types=[
            pltpu.VMEM((NSTEPS, CHUNK), jnp.int32),
            [pltpu.VMEM((CHUNK, EMB_), jnp.float32) for _ in range(NBUF)],
            [pltpu.SemaphoreType.DMA for _ in range(NBUF)],
            [pltpu.SemaphoreType.DMA for _ in range(NBUF)],
        ],
    )(toks, table)
    return out.reshape(tokens.shape[0], tokens.shape[1], EMB_)


# speedup vs baseline: 5.2050x; 1.7557x over previous
import math

import jax
import jax.numpy as jnp
from jax import lax
from jax.experimental import pallas as pl
from jax.experimental.pallas import tpu as pltpu
from jax.experimental.pallas import tpu_sc as plsc

VOCAB_ = 100000
EMB_ = 128
BATCH = 4096
SEQ = 50
NW = 32
BPW = BATCH // NW
NB = 4
NSTEPS = BPW // NB
NBUF = 4
NGRP = NSTEPS // NBUF
SCALE = math.sqrt(EMB_)


def _body(tokens_hbm, table_hbm, out_hbm, idx_v, bufs, gsems, osems):
    nc = 2
    wid = lax.axis_index("s") * nc + lax.axis_index("c")
    obase = wid * BPW

    pltpu.sync_copy(tokens_hbm.at[wid], idx_v)

    def gather(step, j, b):
        return pltpu.make_async_copy(
            table_hbm.at[idx_v.at[step * NB + j]], bufs[b].at[j], gsems[b])

    def writeback(step, b):
        return pltpu.make_async_copy(
            bufs[b], out_hbm.at[pl.ds(obase + step * NB, NB)], osems[b])

    for b in range(NBUF):
        for j in range(NB):
            gather(b, j, b).start()

    def scale_rows(r, buf, j):
        for rr in range(2):
            for k in range(EMB_ // 16):
                sl = pl.ds(k * 16, 16)
                buf[j, 2 * r + rr, sl] = buf[j, 2 * r + rr, sl] * SCALE
        return r

    def grp_body(grp, carry):
        for b in range(NBUF):
            step = grp * NBUF + b
            for j in range(NB):
                gather(step, j, b).wait()
            for j in range(NB):
                lax.fori_loop(
                    0, SEQ // 2, lambda r, _, j=j: scale_rows(r, bufs[b], j), 0)
            writeback(step, b).start()

            @pl.when(grp + 1 < NGRP)
            def _():
                writeback(step, b).wait()
                for j in range(NB):
                    gather(step + NBUF, j, b).start()
        return carry

    lax.fori_loop(0, NGRP, grp_body, 0)

    for b in range(NBUF):
        writeback((NGRP - 1) * NBUF + b, b).wait()


def kernel(tokens, table):
    toks = tokens.reshape(NW, BPW, SEQ).astype(jnp.int32)
    mesh = plsc.VectorSubcoreMesh(core_axis_name="c", subcore_axis_name="s")
    return pl.kernel(
        _body,
        out_type=jax.ShapeDtypeStruct((BATCH, SEQ, EMB_), jnp.float32),
        mesh=mesh,
        scratch_types=[
            pltpu.VMEM((BPW, SEQ), jnp.int32),
            [pltpu.VMEM((NB, SEQ, EMB_), jnp.float32) for _ in range(NBUF)],
            [pltpu.SemaphoreType.DMA for _ in range(NBUF)],
            [pltpu.SemaphoreType.DMA for _ in range(NBUF)],
        ],
    )(toks, table)


# speedup vs baseline: 9.4082x; 1.8075x over previous
import math

import jax
import jax.numpy as jnp
from jax import lax
from jax.experimental import pallas as pl
from jax.experimental.pallas import tpu as pltpu
from jax.experimental.pallas import tpu_sc as plsc

VOCAB_ = 100000
EMB_ = 128
BATCH = 4096
SEQ = 50
NW = 32
BPW = BATCH // NW
NBUF = 5
NGRP = SEQ // NBUF
REFILL_D = 2
SCALE = math.sqrt(EMB_)


def _body(tokens_hbm, table_hbm, out_hbm, idx_v, bufs, gsems, osems):
    nc = 2
    wid = lax.axis_index("s") * nc + lax.axis_index("c")
    bbase = wid * BPW

    pltpu.sync_copy(tokens_hbm.at[:, pl.ds(bbase, BPW)], idx_v)

    def gather(step, b):
        return pltpu.make_async_copy(
            table_hbm.at[idx_v.at[step]], bufs[b], gsems[b])

    def writeback(step, b):
        return pltpu.make_async_copy(
            bufs[b], out_hbm.at[step, pl.ds(bbase, BPW)], osems[b])

    for b in range(NBUF):
        gather(b, b).start()

    def scale_rows(r, buf):
        for rr in range(2):
            for j in range(EMB_ // 16):
                sl = pl.ds(j * 16, 16)
                buf[2 * r + rr, sl] = buf[2 * r + rr, sl] * SCALE
        return r

    def grp_body(grp, carry):
        for b in range(NBUF):
            step = grp * NBUF + b
            rb = (b - REFILL_D) % NBUF
            rstep = step - REFILL_D

            @pl.when((rstep >= 0) & (rstep + NBUF < SEQ))
            def _():
                writeback(rstep, rb).wait()
                gather(rstep + NBUF, rb).start()

            gather(step, b).wait()
            lax.fori_loop(0, BPW // 2, lambda r, _: scale_rows(r, bufs[b]), 0)
            writeback(step, b).start()
        return carry

    lax.fori_loop(0, NGRP, grp_body, 0)

    for b in range(NBUF):
        writeback((NGRP - 1) * NBUF + b, b).wait()


def kernel(tokens, table):
    toks_t = tokens.T.astype(jnp.int32)
    mesh = plsc.VectorSubcoreMesh(core_axis_name="c", subcore_axis_name="s")
    out = pl.kernel(
        _body,
        out_type=jax.ShapeDtypeStruct((SEQ, BATCH, EMB_), jnp.float32),
        mesh=mesh,
        scratch_types=[
            pltpu.VMEM((SEQ, BPW), jnp.int32),
            [pltpu.VMEM((BPW, EMB_), jnp.float32) for _ in range(NBUF)],
            [pltpu.SemaphoreType.DMA for _ in range(NBUF)],
            [pltpu.SemaphoreType.DMA for _ in range(NBUF)],
        ],
    )(toks_t, table)
    return out.transpose(1, 0, 2)
